# Initial kernel scaffold; baseline (speedup 1.0000x reference)
import jax
import jax.numpy as jnp
from jax.experimental import pallas as pl


def kernel(x, edge_index, W1_l, W1_r, b1, W2_l, W2_r, b2, W3, b3):
    raise NotImplementedError("write your pallas kernel here")


# speedup vs baseline: 10.8870x; 10.8870x over previous
import functools

import jax
import jax.numpy as jnp
from jax import lax
from jax.experimental import pallas as pl
from jax.experimental.pallas import tpu as pltpu
from jax.experimental.pallas import tpu_sc as plsc

_N = 10000
_E = 320000
_D_IN = 128
_D = 32

_NC, _NS = 2, 16
_NW = _NC * _NS
_CHUNK = 128
_CPW = 80
_EP = _NW * _CPW * _CHUNK
_NP = 10240
_RPT = _NP // _NS

_ROWBLK = 1280
_NBLK = _NP // _ROWBLK


@functools.lru_cache(maxsize=None)
def _make_sc_segsum(with_count: bool):
    mesh = plsc.VectorSubcoreMesh(core_axis_name="c", subcore_axis_name="s")
    out_type = [jax.ShapeDtypeStruct((_NC * _NP, _D), jnp.float32)]
    scratch = [
        pltpu.VMEM((_CPW, _CHUNK), jnp.int32),
        pltpu.VMEM((_CPW, _CHUNK), jnp.int32),
        pltpu.VMEM((2, _CHUNK, _D), jnp.float32),
        pltpu.VMEM_SHARED((_NP, _D), jnp.float32),
        pltpu.SemaphoreType.DMA,
        pltpu.SemaphoreType.DMA,
    ]
    if with_count:
        out_type.append(jax.ShapeDtypeStruct((_NC * _NP,), jnp.float32))
        scratch += [
            pltpu.VMEM((_CHUNK,), jnp.float32),
            pltpu.VMEM((_CHUNK,), jnp.float32),
            pltpu.VMEM_SHARED((_NP,), jnp.float32),
        ]

    def body(y_hbm, src_hbm, dst_hbm, *rest):
        if with_count:
            (agg_out, cnt_out, src_v, dst_v, rows_v, agg_sh, sem0, sem1,
             one_v, zc_v, cnt_sh) = rest
        else:
            agg_out, src_v, dst_v, rows_v, agg_sh, sem0, sem1 = rest

        c = lax.axis_index("c")
        s = lax.axis_index("s")
        w = s * _NC + c

        pltpu.sync_copy(src_hbm.at[pl.ds(w * _CPW, _CPW)], src_v)
        pltpu.sync_copy(dst_hbm.at[pl.ds(w * _CPW, _CPW)], dst_v)

        zv = jnp.zeros((16,), jnp.float32)

        def fill_rows(i, _):
            rows_v[0, i, pl.ds(0, 16)] = zv
            rows_v[0, i, pl.ds(16, 16)] = zv
            return 0
        lax.fori_loop(0, _CHUNK, fill_rows, 0)

        if with_count:
            ov = jnp.ones((16,), jnp.float32)

            def fill_small(i, _):
                one_v[pl.ds(i * 16, 16)] = ov
                zc_v[pl.ds(i * 16, 16)] = zv
                return 0
            lax.fori_loop(0, _CHUNK // 16, fill_small, 0)

        for k in range(_RPT // _CHUNK):
            base = s * _RPT + k * _CHUNK
            pltpu.sync_copy(rows_v.at[0], agg_sh.at[pl.ds(base, _CHUNK)])
            if with_count:
                pltpu.sync_copy(zc_v, cnt_sh.at[pl.ds(base, _CHUNK)])
        plsc.subcore_barrier()

        def gather(j, buf, sem):
            return pltpu.async_copy(y_hbm.at[src_v.at[j]], rows_v.at[buf], sem)

        def drain(buf, sem):
            pltpu.make_async_copy(y_hbm.at[src_v.at[0]], rows_v.at[buf], sem).wait()

        def scat(j, buf):
            pltpu.sync_copy(rows_v.at[buf], agg_sh.at[dst_v.at[j]], add=True)
            if with_count:
                pltpu.sync_copy(one_v, cnt_sh.at[dst_v.at[j]], add=True)

        gather(0, 0, sem0)

        def eloop(i, _):
            j0 = i * 2
            gather(j0 + 1, 1, sem1)
            drain(0, sem0)
            scat(j0, 0)

            @pl.when(i + 1 < _CPW // 2)
            def _():
                gather(j0 + 2, 0, sem0)
            drain(1, sem1)
            scat(j0 + 1, 1)
            return 0
        lax.fori_loop(0, _CPW // 2, eloop, 0)
        plsc.subcore_barrier()

        ob = c * _NP + s * _RPT
        pltpu.sync_copy(agg_sh.at[pl.ds(s * _RPT, _RPT)],
                        agg_out.at[pl.ds(ob, _RPT)])
        if with_count:
            pltpu.sync_copy(cnt_sh.at[pl.ds(s * _RPT, _RPT)],
                            cnt_out.at[pl.ds(ob, _RPT)])

    return pl.kernel(
        body, out_type=out_type, mesh=mesh, scratch_types=scratch,
        compiler_params=pltpu.CompilerParams(use_tc_tiling_on_sc=False))


def _tc1_body(x_ref, w_ref, b_ref, y_ref, r_ref):
    xw = jnp.dot(x_ref[...], w_ref[...], preferred_element_type=jnp.float32)
    y_ref[...] = xw[:, :_D]
    r_ref[...] = xw[:, _D:] + b_ref[...]


def _tc2_body(aggp_ref, cntp_ref, r1_ref, w_ref, b_ref, y2_ref, r2_ref):
    agg = aggp_ref[0] + aggp_ref[1]
    inv = 1.0 / jnp.maximum(cntp_ref[0] + cntp_ref[1], 1.0)
    h = jnp.maximum(agg * inv + r1_ref[...], 0.0)
    hw = jnp.dot(h, w_ref[...], preferred_element_type=jnp.float32)
    y2_ref[...] = hw[:, :_D]
    r2_ref[...] = hw[:, _D:] + b_ref[...]


def _tc3_body(aggp_ref, cntp_ref, r2_ref, w_ref, b_ref, o_ref):
    agg = aggp_ref[0] + aggp_ref[1]
    inv = 1.0 / jnp.maximum(cntp_ref[0] + cntp_ref[1], 1.0)
    h2 = agg * inv + r2_ref[...]
    o_ref[...] = jnp.dot(h2, w_ref[...],
                         preferred_element_type=jnp.float32) + b_ref[...]


def _rows_spec(width):
    return pl.BlockSpec((_ROWBLK, width), lambda i: (i, 0))


def _part_spec(width):
    return pl.BlockSpec((2, _ROWBLK, width), lambda i: (0, i, 0))


def _full_spec(shape):
    return pl.BlockSpec(shape, lambda i: tuple(0 for _ in shape))


_tc1 = pl.pallas_call(
    _tc1_body,
    grid=(_NBLK,),
    in_specs=[_rows_spec(_D_IN), _full_spec((_D_IN, 2 * _D)), _full_spec((1, _D))],
    out_specs=[_rows_spec(_D), _rows_spec(_D)],
    out_shape=[jax.ShapeDtypeStruct((_NP, _D), jnp.float32)] * 2,
)

_tc2 = pl.pallas_call(
    _tc2_body,
    grid=(_NBLK,),
    in_specs=[_part_spec(_D), _part_spec(1), _rows_spec(_D),
              _full_spec((_D, 2 * _D)), _full_spec((1, _D))],
    out_specs=[_rows_spec(_D), _rows_spec(_D)],
    out_shape=[jax.ShapeDtypeStruct((_NP, _D), jnp.float32)] * 2,
)

_tc3 = pl.pallas_call(
    _tc3_body,
    grid=(_NBLK,),
    in_specs=[_part_spec(_D), _part_spec(1), _rows_spec(_D),
              _full_spec((_D, 1)), _full_spec((1, 1))],
    out_specs=_rows_spec(1),
    out_shape=jax.ShapeDtypeStruct((_NP, 1), jnp.float32),
)


def kernel(x, edge_index, W1_l, W1_r, b1, W2_l, W2_r, b2, W3, b3):
    src = edge_index[0].astype(jnp.int32)
    dst = edge_index[1].astype(jnp.int32)
    pad = _EP - _E
    src2d = jnp.concatenate([src, jnp.zeros((pad,), jnp.int32)]
                            ).reshape(_EP // _CHUNK, _CHUNK)
    dst2d = jnp.concatenate([dst, jnp.full((pad,), _N, jnp.int32)]
                            ).reshape(_EP // _CHUNK, _CHUNK)
    x_p = jnp.pad(x, ((0, _NP - _N), (0, 0)))

    W1 = jnp.concatenate([W1_l, W1_r], axis=1)
    W2 = jnp.concatenate([W2_l, W2_r], axis=1)

    y1, r1 = _tc1(x_p, W1, b1.reshape(1, _D))
    agg1, cnt = _make_sc_segsum(True)(y1, src2d, dst2d)
    agg1 = agg1.reshape(_NC, _NP, _D)
    cnt3 = cnt.reshape(_NC, _NP, 1)
    y2, r2 = _tc2(agg1, cnt3, r1, W2, b2.reshape(1, _D))
    (agg2,) = jax.tree.leaves(_make_sc_segsum(False)(y2, src2d, dst2d))
    agg2 = agg2.reshape(_NC, _NP, _D)
    out = _tc3(agg2, cnt3, r2, W3, b3.reshape(1, 1))
    return out[:_N]


# speedup vs baseline: 10.8984x; 1.0010x over previous
import functools

import jax
import jax.numpy as jnp
from jax import lax
from jax.experimental import pallas as pl
from jax.experimental.pallas import tpu as pltpu
from jax.experimental.pallas import tpu_sc as plsc

_N = 10000
_E = 320000
_D_IN = 128
_D = 32

_NC, _NS = 2, 16
_NW = _NC * _NS
_CHUNK = 128
_CPW = 80
_EP = _NW * _CPW * _CHUNK
_NP = 10240
_RPT = _NP // _NS

_ROWBLK = 1280
_NBLK = _NP // _ROWBLK


_NBUF = 10
_LOOKA = 7


@functools.lru_cache(maxsize=None)
def _make_sc_segsum(with_count: bool):
    mesh = plsc.VectorSubcoreMesh(core_axis_name="c", subcore_axis_name="s")
    out_type = [jax.ShapeDtypeStruct((_NC * _NP, _D), jnp.float32)]
    scratch = [
        pltpu.VMEM((_CPW, _CHUNK), jnp.int32),
        pltpu.VMEM((_CPW, _CHUNK), jnp.int32),
        pltpu.VMEM((_NBUF, _CHUNK, _D), jnp.float32),
        pltpu.VMEM_SHARED((_NP, _D), jnp.float32),
        [pltpu.SemaphoreType.DMA] * _NBUF,
        [pltpu.SemaphoreType.DMA] * _NBUF,
    ]
    if with_count:
        out_type.append(jax.ShapeDtypeStruct((_NC * _NP,), jnp.float32))
        scratch += [
            pltpu.VMEM((_CHUNK,), jnp.float32),
            pltpu.VMEM((_CHUNK,), jnp.float32),
            pltpu.VMEM_SHARED((_NP,), jnp.float32),
        ]

    def body(y_hbm, src_hbm, dst_hbm, *rest):
        if with_count:
            (agg_out, cnt_out, src_v, dst_v, rows_v, agg_sh, gsem, ssem,
             one_v, zc_v, cnt_sh) = rest
        else:
            agg_out, src_v, dst_v, rows_v, agg_sh, gsem, ssem = rest

        c = lax.axis_index("c")
        s = lax.axis_index("s")
        w = s * _NC + c

        pltpu.sync_copy(src_hbm.at[pl.ds(w * _CPW, _CPW)], src_v)
        pltpu.sync_copy(dst_hbm.at[pl.ds(w * _CPW, _CPW)], dst_v)

        zv = jnp.zeros((16,), jnp.float32)

        def fill_rows(i, _):
            rows_v[0, i, pl.ds(0, 16)] = zv
            rows_v[0, i, pl.ds(16, 16)] = zv
            return 0
        lax.fori_loop(0, _CHUNK, fill_rows, 0)

        if with_count:
            ov = jnp.ones((16,), jnp.float32)

            def fill_small(i, _):
                one_v[pl.ds(i * 16, 16)] = ov
                zc_v[pl.ds(i * 16, 16)] = zv
                return 0
            lax.fori_loop(0, _CHUNK // 16, fill_small, 0)

        for k in range(_RPT // _CHUNK):
            base = s * _RPT + k * _CHUNK
            pltpu.sync_copy(rows_v.at[0], agg_sh.at[pl.ds(base, _CHUNK)])
            if with_count:
                pltpu.sync_copy(zc_v, cnt_sh.at[pl.ds(base, _CHUNK)])
        plsc.subcore_barrier()

        def gather(j, b):
            pltpu.async_copy(y_hbm.at[src_v.at[j]], rows_v.at[b], gsem[b])

        def wait_gather(b):
            pltpu.make_async_copy(
                y_hbm.at[src_v.at[0]], rows_v.at[b], gsem[b]).wait()

        def scatter(j, b):
            pltpu.async_copy(rows_v.at[b], agg_sh.at[dst_v.at[j]], ssem[b],
                             add=True)
            if with_count:
                pltpu.async_copy(one_v, cnt_sh.at[dst_v.at[j]], ssem[b],
                                 add=True)

        def wait_scatter(b):
            pltpu.make_async_copy(
                rows_v.at[b], agg_sh.at[dst_v.at[0]], ssem[b]).wait()
            if with_count:
                pltpu.make_async_copy(
                    one_v, cnt_sh.at[dst_v.at[0]], ssem[b]).wait()

        lag = _NBUF - _LOOKA
        npb = _CPW // _NBUF

        for b in range(_LOOKA):
            gather(b, b)

        def eloop(i, _):
            for b in range(_NBUF):
                j = i * _NBUF + b
                bf = (b + _LOOKA) % _NBUF
                if b >= lag:
                    wait_scatter(bf)
                else:
                    @pl.when(i > 0)
                    def _(bf=bf):
                        wait_scatter(bf)
                if b < (_CPW - _LOOKA) % _NBUF:
                    gather(j + _LOOKA, bf)
                else:
                    @pl.when(i + 1 < npb)
                    def _(j=j, bf=bf):
                        gather(j + _LOOKA, bf)
                wait_gather(b)
                scatter(j, b)
            return 0
        lax.fori_loop(0, npb, eloop, 0)
        for b in range(lag):
            wait_scatter((_CPW - lag + b) % _NBUF)
        plsc.subcore_barrier()

        ob = c * _NP + s * _RPT
        pltpu.sync_copy(agg_sh.at[pl.ds(s * _RPT, _RPT)],
                        agg_out.at[pl.ds(ob, _RPT)])
        if with_count:
            pltpu.sync_copy(cnt_sh.at[pl.ds(s * _RPT, _RPT)],
                            cnt_out.at[pl.ds(ob, _RPT)])

    return pl.kernel(
        body, out_type=out_type, mesh=mesh, scratch_types=scratch,
        compiler_params=pltpu.CompilerParams(use_tc_tiling_on_sc=False))


def _tc1_body(x_ref, w_ref, b_ref, y_ref, r_ref):
    xw = jnp.dot(x_ref[...], w_ref[...], preferred_element_type=jnp.float32)
    y_ref[...] = xw[:, :_D]
    r_ref[...] = xw[:, _D:] + b_ref[...]


def _tc2_body(aggp_ref, cntp_ref, r1_ref, w_ref, b_ref, y2_ref, r2_ref):
    agg = aggp_ref[0] + aggp_ref[1]
    inv = 1.0 / jnp.maximum(cntp_ref[0] + cntp_ref[1], 1.0)
    h = jnp.maximum(agg * inv + r1_ref[...], 0.0)
    hw = jnp.dot(h, w_ref[...], preferred_element_type=jnp.float32)
    y2_ref[...] = hw[:, :_D]
    r2_ref[...] = hw[:, _D:] + b_ref[...]


def _tc3_body(aggp_ref, cntp_ref, r2_ref, w_ref, b_ref, o_ref):
    agg = aggp_ref[0] + aggp_ref[1]
    inv = 1.0 / jnp.maximum(cntp_ref[0] + cntp_ref[1], 1.0)
    h2 = agg * inv + r2_ref[...]
    o_ref[...] = jnp.dot(h2, w_ref[...],
                         preferred_element_type=jnp.float32) + b_ref[...]


def _rows_spec(width):
    return pl.BlockSpec((_ROWBLK, width), lambda i: (i, 0))


def _part_spec(width):
    return pl.BlockSpec((2, _ROWBLK, width), lambda i: (0, i, 0))


def _full_spec(shape):
    return pl.BlockSpec(shape, lambda i: tuple(0 for _ in shape))


_tc1 = pl.pallas_call(
    _tc1_body,
    grid=(_NBLK,),
    in_specs=[_rows_spec(_D_IN), _full_spec((_D_IN, 2 * _D)), _full_spec((1, _D))],
    out_specs=[_rows_spec(_D), _rows_spec(_D)],
    out_shape=[jax.ShapeDtypeStruct((_NP, _D), jnp.float32)] * 2,
)

_tc2 = pl.pallas_call(
    _tc2_body,
    grid=(_NBLK,),
    in_specs=[_part_spec(_D), _part_spec(1), _rows_spec(_D),
              _full_spec((_D, 2 * _D)), _full_spec((1, _D))],
    out_specs=[_rows_spec(_D), _rows_spec(_D)],
    out_shape=[jax.ShapeDtypeStruct((_NP, _D), jnp.float32)] * 2,
)

_tc3 = pl.pallas_call(
    _tc3_body,
    grid=(_NBLK,),
    in_specs=[_part_spec(_D), _part_spec(1), _rows_spec(_D),
              _full_spec((_D, 1)), _full_spec((1, 1))],
    out_specs=_rows_spec(1),
    out_shape=jax.ShapeDtypeStruct((_NP, 1), jnp.float32),
)


def kernel(x, edge_index, W1_l, W1_r, b1, W2_l, W2_r, b2, W3, b3):
    src = edge_index[0].astype(jnp.int32)
    dst = edge_index[1].astype(jnp.int32)
    pad = _EP - _E
    src2d = jnp.concatenate([src, jnp.zeros((pad,), jnp.int32)]
                            ).reshape(_EP // _CHUNK, _CHUNK)
    dst2d = jnp.concatenate([dst, jnp.full((pad,), _N, jnp.int32)]
                            ).reshape(_EP // _CHUNK, _CHUNK)
    x_p = jnp.pad(x, ((0, _NP - _N), (0, 0)))

    W1 = jnp.concatenate([W1_l, W1_r], axis=1)
    W2 = jnp.concatenate([W2_l, W2_r], axis=1)

    y1, r1 = _tc1(x_p, W1, b1.reshape(1, _D))
    agg1, cnt = _make_sc_segsum(True)(y1, src2d, dst2d)
    agg1 = agg1.reshape(_NC, _NP, _D)
    cnt3 = cnt.reshape(_NC, _NP, 1)
    y2, r2 = _tc2(agg1, cnt3, r1, W2, b2.reshape(1, _D))
    (agg2,) = jax.tree.leaves(_make_sc_segsum(False)(y2, src2d, dst2d))
    agg2 = agg2.reshape(_NC, _NP, _D)
    out = _tc3(agg2, cnt3, r2, W3, b3.reshape(1, 1))
    return out[:_N]


# speedup vs baseline: 18.6849x; 1.7145x over previous
import functools

import jax
import jax.numpy as jnp
from jax import lax
from jax.experimental import pallas as pl
from jax.experimental.pallas import tpu as pltpu
from jax.experimental.pallas import tpu_sc as plsc

_N = 10000
_E = 320000
_D_IN = 128
_D = 32

_NC, _NS = 2, 16
_NW = _NC * _NS
_CHUNK = 128
_CPW = 80
_EP = _NW * _CPW * _CHUNK
_NP = 10240
_RPT = _NP // _NS

_ROWBLK = 1280
_NBLK = _NP // _ROWBLK


_NBUF = 10
_LOOKA = 7


@functools.lru_cache(maxsize=None)
def _make_sc_segsum(with_count: bool):
    mesh = plsc.VectorSubcoreMesh(core_axis_name="c", subcore_axis_name="s")
    out_type = [jax.ShapeDtypeStruct((_NC * _NP, _D), jnp.float32)]
    scratch = [
        pltpu.VMEM((_CPW, _CHUNK), jnp.int32),
        pltpu.VMEM((_CPW, _CHUNK), jnp.int32),
        pltpu.VMEM((_NBUF, _CHUNK, _D), jnp.float32),
        pltpu.VMEM_SHARED((_NP, _D), jnp.float32),
        pltpu.VMEM_SHARED((_NP, _D), jnp.float32),
        [pltpu.SemaphoreType.DMA] * _NBUF,
        [pltpu.SemaphoreType.DMA] * _NBUF,
    ]
    if with_count:
        out_type.append(jax.ShapeDtypeStruct((_NC * _NP,), jnp.float32))
        scratch += [
            pltpu.VMEM((_CHUNK,), jnp.float32),
            pltpu.VMEM((_CHUNK,), jnp.float32),
            pltpu.VMEM_SHARED((_NP,), jnp.float32),
        ]

    def body(y_hbm, src_hbm, dst_hbm, *rest):
        if with_count:
            (agg_out, cnt_out, src_v, dst_v, rows_v, agg_sh, y_sh, gsem, ssem,
             one_v, zc_v, cnt_sh) = rest
        else:
            agg_out, src_v, dst_v, rows_v, agg_sh, y_sh, gsem, ssem = rest

        c = lax.axis_index("c")
        s = lax.axis_index("s")
        w = s * _NC + c

        pltpu.sync_copy(src_hbm.at[pl.ds(w * _CPW, _CPW)], src_v)
        pltpu.sync_copy(dst_hbm.at[pl.ds(w * _CPW, _CPW)], dst_v)
        pltpu.sync_copy(y_hbm.at[pl.ds(s * _RPT, _RPT)],
                        y_sh.at[pl.ds(s * _RPT, _RPT)])

        zv = jnp.zeros((16,), jnp.float32)

        def fill_rows(i, _):
            rows_v[0, i, pl.ds(0, 16)] = zv
            rows_v[0, i, pl.ds(16, 16)] = zv
            return 0
        lax.fori_loop(0, _CHUNK, fill_rows, 0)

        if with_count:
            ov = jnp.ones((16,), jnp.float32)

            def fill_small(i, _):
                one_v[pl.ds(i * 16, 16)] = ov
                zc_v[pl.ds(i * 16, 16)] = zv
                return 0
            lax.fori_loop(0, _CHUNK // 16, fill_small, 0)

        for k in range(_RPT // _CHUNK):
            base = s * _RPT + k * _CHUNK
            pltpu.sync_copy(rows_v.at[0], agg_sh.at[pl.ds(base, _CHUNK)])
            if with_count:
                pltpu.sync_copy(zc_v, cnt_sh.at[pl.ds(base, _CHUNK)])
        plsc.subcore_barrier()

        def gather(j, b):
            pltpu.async_copy(y_sh.at[src_v.at[j]], rows_v.at[b], gsem[b])

        def wait_gather(b):
            pltpu.make_async_copy(
                y_sh.at[src_v.at[0]], rows_v.at[b], gsem[b]).wait()

        def scatter(j, b):
            pltpu.async_copy(rows_v.at[b], agg_sh.at[dst_v.at[j]], ssem[b],
                             add=True)
            if with_count:
                pltpu.async_copy(one_v, cnt_sh.at[dst_v.at[j]], ssem[b],
                                 add=True)

        def wait_scatter(b):
            pltpu.make_async_copy(
                rows_v.at[b], agg_sh.at[dst_v.at[0]], ssem[b]).wait()
            if with_count:
                pltpu.make_async_copy(
                    one_v, cnt_sh.at[dst_v.at[0]], ssem[b]).wait()

        lag = _NBUF - _LOOKA
        npb = _CPW // _NBUF

        for b in range(_LOOKA):
            gather(b, b)

        def eloop(i, _):
            for b in range(_NBUF):
                j = i * _NBUF + b
                bf = (b + _LOOKA) % _NBUF
                if b >= lag:
                    wait_scatter(bf)
                else:
                    @pl.when(i > 0)
                    def _(bf=bf):
                        wait_scatter(bf)
                if b < (_CPW - _LOOKA) % _NBUF:
                    gather(j + _LOOKA, bf)
                else:
                    @pl.when(i + 1 < npb)
                    def _(j=j, bf=bf):
                        gather(j + _LOOKA, bf)
                wait_gather(b)
                scatter(j, b)
            return 0
        lax.fori_loop(0, npb, eloop, 0)
        for b in range(lag):
            wait_scatter((_CPW - lag + b) % _NBUF)
        plsc.subcore_barrier()

        ob = c * _NP + s * _RPT
        pltpu.sync_copy(agg_sh.at[pl.ds(s * _RPT, _RPT)],
                        agg_out.at[pl.ds(ob, _RPT)])
        if with_count:
            pltpu.sync_copy(cnt_sh.at[pl.ds(s * _RPT, _RPT)],
                            cnt_out.at[pl.ds(ob, _RPT)])

    return pl.kernel(
        body, out_type=out_type, mesh=mesh, scratch_types=scratch,
        compiler_params=pltpu.CompilerParams(use_tc_tiling_on_sc=False))


def _tc1_body(x_ref, w_ref, b_ref, y_ref, r_ref):
    xw = jnp.dot(x_ref[...], w_ref[...], preferred_element_type=jnp.float32)
    y_ref[...] = xw[:, :_D]
    r_ref[...] = xw[:, _D:] + b_ref[...]


def _tc2_body(aggp_ref, cntp_ref, r1_ref, w_ref, b_ref, y2_ref, r2_ref):
    agg = aggp_ref[0] + aggp_ref[1]
    inv = 1.0 / jnp.maximum(cntp_ref[0] + cntp_ref[1], 1.0)
    h = jnp.maximum(agg * inv + r1_ref[...], 0.0)
    hw = jnp.dot(h, w_ref[...], preferred_element_type=jnp.float32)
    y2_ref[...] = hw[:, :_D]
    r2_ref[...] = hw[:, _D:] + b_ref[...]


def _tc3_body(aggp_ref, cntp_ref, r2_ref, w_ref, b_ref, o_ref):
    agg = aggp_ref[0] + aggp_ref[1]
    inv = 1.0 / jnp.maximum(cntp_ref[0] + cntp_ref[1], 1.0)
    h2 = agg * inv + r2_ref[...]
    o_ref[...] = jnp.dot(h2, w_ref[...],
                         preferred_element_type=jnp.float32) + b_ref[...]


def _rows_spec(width):
    return pl.BlockSpec((_ROWBLK, width), lambda i: (i, 0))


def _part_spec(width):
    return pl.BlockSpec((2, _ROWBLK, width), lambda i: (0, i, 0))


def _full_spec(shape):
    return pl.BlockSpec(shape, lambda i: tuple(0 for _ in shape))


_tc1 = pl.pallas_call(
    _tc1_body,
    grid=(_NBLK,),
    in_specs=[_rows_spec(_D_IN), _full_spec((_D_IN, 2 * _D)), _full_spec((1, _D))],
    out_specs=[_rows_spec(_D), _rows_spec(_D)],
    out_shape=[jax.ShapeDtypeStruct((_NP, _D), jnp.float32)] * 2,
)

_tc2 = pl.pallas_call(
    _tc2_body,
    grid=(_NBLK,),
    in_specs=[_part_spec(_D), _part_spec(1), _rows_spec(_D),
              _full_spec((_D, 2 * _D)), _full_spec((1, _D))],
    out_specs=[_rows_spec(_D), _rows_spec(_D)],
    out_shape=[jax.ShapeDtypeStruct((_NP, _D), jnp.float32)] * 2,
)

_tc3 = pl.pallas_call(
    _tc3_body,
    grid=(_NBLK,),
    in_specs=[_part_spec(_D), _part_spec(1), _rows_spec(_D),
              _full_spec((_D, 1)), _full_spec((1, 1))],
    out_specs=_rows_spec(1),
    out_shape=jax.ShapeDtypeStruct((_NP, 1), jnp.float32),
)


def kernel(x, edge_index, W1_l, W1_r, b1, W2_l, W2_r, b2, W3, b3):
    src = edge_index[0].astype(jnp.int32)
    dst = edge_index[1].astype(jnp.int32)
    pad = _EP - _E
    src2d = jnp.concatenate([src, jnp.zeros((pad,), jnp.int32)]
                            ).reshape(_EP // _CHUNK, _CHUNK)
    dst2d = jnp.concatenate([dst, jnp.full((pad,), _N, jnp.int32)]
                            ).reshape(_EP // _CHUNK, _CHUNK)
    x_p = jnp.pad(x, ((0, _NP - _N), (0, 0)))

    W1 = jnp.concatenate([W1_l, W1_r], axis=1)
    W2 = jnp.concatenate([W2_l, W2_r], axis=1)

    y1, r1 = _tc1(x_p, W1, b1.reshape(1, _D))
    agg1, cnt = _make_sc_segsum(True)(y1, src2d, dst2d)
    agg1 = agg1.reshape(_NC, _NP, _D)
    cnt3 = cnt.reshape(_NC, _NP, 1)
    y2, r2 = _tc2(agg1, cnt3, r1, W2, b2.reshape(1, _D))
    (agg2,) = jax.tree.leaves(_make_sc_segsum(False)(y2, src2d, dst2d))
    agg2 = agg2.reshape(_NC, _NP, _D)
    out = _tc3(agg2, cnt3, r2, W3, b3.reshape(1, 1))
    return out[:_N]


# speedup vs baseline: 21.5111x; 1.1513x over previous
import functools

import jax
import jax.numpy as jnp
from jax import lax
from jax.scipy.linalg import block_diag
from jax.experimental import pallas as pl
from jax.experimental.pallas import tpu as pltpu
from jax.experimental.pallas import tpu_sc as plsc

_N = 10000
_E = 320000
_D_IN = 128
_D = 32

_NC, _NS = 2, 16
_NW = _NC * _NS
_CHUNK = 128
_CPW = 80
_EP = _NW * _CPW * _CHUNK
_NP = 10240
_RPT = _NP // _NS
_PR = _NP // 4
_CR = _NP // 128

_PBLK = _PR // 8
_NBLK = 8

_NBUF = 10
_LOOKA = 7


@functools.lru_cache(maxsize=None)
def _make_sc_segsum(with_count: bool):
    mesh = plsc.VectorSubcoreMesh(core_axis_name="c", subcore_axis_name="s")
    out_type = [jax.ShapeDtypeStruct((_NC * _NP, _D), jnp.float32)]
    scratch = [
        pltpu.VMEM((_CPW, _CHUNK), jnp.int32),
        pltpu.VMEM((_CPW, _CHUNK), jnp.int32),
        pltpu.VMEM((_NBUF, _CHUNK, _D), jnp.float32),
        pltpu.VMEM_SHARED((_NP, _D), jnp.float32),
        pltpu.VMEM_SHARED((_NP, _D), jnp.float32),
        [pltpu.SemaphoreType.DMA] * _NBUF,
        [pltpu.SemaphoreType.DMA] * _NBUF,
    ]
    if with_count:
        out_type.append(jax.ShapeDtypeStruct((_NC * _NP, _D), jnp.float32))
        scratch += [
            pltpu.VMEM((_CHUNK, _D), jnp.float32),
            pltpu.VMEM_SHARED((_NP, _D), jnp.float32),
        ]

    def body(y_hbm, src_hbm, dst_hbm, *rest):
        if with_count:
            (agg_out, cnt_out, src_v, dst_v, rows_v, agg_sh, y_sh, gsem, ssem,
             one_v, cnt_sh) = rest
        else:
            agg_out, src_v, dst_v, rows_v, agg_sh, y_sh, gsem, ssem = rest

        c = lax.axis_index("c")
        s = lax.axis_index("s")
        w = s * _NC + c

        pltpu.sync_copy(src_hbm.at[pl.ds(w * _CPW, _CPW)], src_v)
        pltpu.sync_copy(dst_hbm.at[pl.ds(w * _CPW, _CPW)], dst_v)
        pltpu.sync_copy(y_hbm.at[pl.ds(s * _RPT, _RPT)],
                        y_sh.at[pl.ds(s * _RPT, _RPT)])

        zv = jnp.zeros((16,), jnp.float32)

        def fill_rows(i, _):
            rows_v[0, i, pl.ds(0, 16)] = zv
            rows_v[0, i, pl.ds(16, 16)] = zv
            return 0
        lax.fori_loop(0, _CHUNK, fill_rows, 0)

        if with_count:
            ov = jnp.ones((16,), jnp.float32)

            def fill_ones(i, _):
                one_v[i, pl.ds(0, 16)] = ov
                one_v[i, pl.ds(16, 16)] = ov
                return 0
            lax.fori_loop(0, _CHUNK, fill_ones, 0)

        for k in range(_RPT // _CHUNK):
            base = s * _RPT + k * _CHUNK
            pltpu.sync_copy(rows_v.at[0], agg_sh.at[pl.ds(base, _CHUNK)])
            if with_count:
                pltpu.sync_copy(rows_v.at[0], cnt_sh.at[pl.ds(base, _CHUNK)])
        plsc.subcore_barrier()

        def gather(j, b):
            pltpu.async_copy(y_sh.at[src_v.at[j]], rows_v.at[b], gsem[b])

        def wait_gather(b):
            pltpu.make_async_copy(
                y_sh.at[src_v.at[0]], rows_v.at[b], gsem[b]).wait()

        def scatter(j, b):
            pltpu.async_copy(rows_v.at[b], agg_sh.at[dst_v.at[j]], ssem[b],
                             add=True)
            if with_count:
                pltpu.async_copy(one_v, cnt_sh.at[dst_v.at[j]], ssem[b],
                                 add=True)

        def wait_scatter(b):
            pltpu.make_async_copy(
                rows_v.at[b], agg_sh.at[dst_v.at[0]], ssem[b]).wait()
            if with_count:
                pltpu.make_async_copy(
                    one_v, cnt_sh.at[dst_v.at[0]], ssem[b]).wait()

        lag = _NBUF - _LOOKA
        npb = _CPW // _NBUF

        for b in range(_LOOKA):
            gather(b, b)

        def eloop(i, _):
            for b in range(_NBUF):
                j = i * _NBUF + b
                bf = (b + _LOOKA) % _NBUF
                if b >= lag:
                    wait_scatter(bf)
                else:
                    @pl.when(i > 0)
                    def _(bf=bf):
                        wait_scatter(bf)
                if b < (_CPW - _LOOKA) % _NBUF:
                    gather(j + _LOOKA, bf)
                else:
                    @pl.when(i + 1 < npb)
                    def _(j=j, bf=bf):
                        gather(j + _LOOKA, bf)
                wait_gather(b)
                scatter(j, b)
            return 0
        lax.fori_loop(0, npb, eloop, 0)
        for b in range(lag):
            wait_scatter((_CPW - lag + b) % _NBUF)
        plsc.subcore_barrier()

        pltpu.sync_copy(agg_sh.at[pl.ds(s * _RPT, _RPT)],
                        agg_out.at[pl.ds(c * _NP + s * _RPT, _RPT)])
        if with_count:
            pltpu.sync_copy(cnt_sh.at[pl.ds(s * _RPT, _RPT)],
                            cnt_out.at[pl.ds(c * _NP + s * _RPT, _RPT)])

    return pl.kernel(
        body, out_type=out_type, mesh=mesh, scratch_types=scratch,
        compiler_params=pltpu.CompilerParams(use_tc_tiling_on_sc=False))


def _tc1_body(x_ref, wl_ref, wr_ref, b_ref, y_ref, r_ref):
    xb = x_ref[...]
    y_ref[...] = jnp.dot(xb, wl_ref[...], preferred_element_type=jnp.float32)
    r_ref[...] = jnp.dot(xb, wr_ref[...],
                         preferred_element_type=jnp.float32) + b_ref[...]


def _inv_packed(cntp_ref):
    return 1.0 / jnp.maximum(cntp_ref[0] + cntp_ref[1], 1.0)


def _tc2_body(aggp_ref, cntp_ref, r1_ref, wl_ref, wr_ref, b_ref,
              y2_ref, r2_ref):
    agg = aggp_ref[0] + aggp_ref[1]
    h = jnp.maximum(agg * _inv_packed(cntp_ref) + r1_ref[...], 0.0)
    y2_ref[...] = jnp.dot(h, wl_ref[...], preferred_element_type=jnp.float32)
    r2_ref[...] = jnp.dot(h, wr_ref[...],
                          preferred_element_type=jnp.float32) + b_ref[...]


def _tc3_body(aggp_ref, cntp_ref, r2_ref, w_ref, b_ref, o_ref):
    agg = aggp_ref[0] + aggp_ref[1]
    h2 = agg * _inv_packed(cntp_ref) + r2_ref[...]
    o_ref[...] = jnp.dot(h2, w_ref[...],
                         preferred_element_type=jnp.float32) + b_ref[...]


def _rows_spec(width):
    return pl.BlockSpec((_PBLK, width), lambda i: (i, 0))


def _part_spec(rows, width):
    return pl.BlockSpec((2, rows, width), lambda i: (0, i, 0))


def _full_spec(shape):
    return pl.BlockSpec(shape, lambda i: tuple(0 for _ in shape))


_CBLK = _CR // _NBLK

_tc1 = pl.pallas_call(
    _tc1_body,
    grid=(_NBLK,),
    in_specs=[pl.BlockSpec((_PBLK, 4 * _D_IN), lambda i: (i, 0)),
              _full_spec((4 * _D_IN, 128)), _full_spec((4 * _D_IN, 128)),
              _full_spec((1, 128))],
    out_specs=[_rows_spec(128), _rows_spec(128)],
    out_shape=[jax.ShapeDtypeStruct((_PR, 128), jnp.float32)] * 2,
)

_tc2 = pl.pallas_call(
    _tc2_body,
    grid=(_NBLK,),
    in_specs=[_part_spec(_PBLK, 128), _part_spec(_PBLK, 128),
              _rows_spec(128), _full_spec((128, 128)), _full_spec((128, 128)),
              _full_spec((1, 128))],
    out_specs=[_rows_spec(128), _rows_spec(128)],
    out_shape=[jax.ShapeDtypeStruct((_PR, 128), jnp.float32)] * 2,
)

_tc3 = pl.pallas_call(
    _tc3_body,
    grid=(_NBLK,),
    in_specs=[_part_spec(_PBLK, 128), _part_spec(_PBLK, 128),
              _rows_spec(128), _full_spec((128, 4)), _full_spec((1, 1))],
    out_specs=_rows_spec(4),
    out_shape=jax.ShapeDtypeStruct((_PR, 4), jnp.float32),
)


def kernel(x, edge_index, W1_l, W1_r, b1, W2_l, W2_r, b2, W3, b3):
    src = edge_index[0].astype(jnp.int32)
    dst = edge_index[1].astype(jnp.int32)
    pad = _EP - _E
    src2d = jnp.concatenate([src, jnp.zeros((pad,), jnp.int32)]
                            ).reshape(_EP // _CHUNK, _CHUNK)
    dst2d = jnp.concatenate([dst, jnp.full((pad,), _N, jnp.int32)]
                            ).reshape(_EP // _CHUNK, _CHUNK)
    xp = jnp.pad(x, ((0, _NP - _N), (0, 0))).reshape(_PR, 4 * _D_IN)

    W1lB = block_diag(W1_l, W1_l, W1_l, W1_l)
    W1rB = block_diag(W1_r, W1_r, W1_r, W1_r)
    W2lB = block_diag(W2_l, W2_l, W2_l, W2_l)
    W2rB = block_diag(W2_r, W2_r, W2_r, W2_r)
    W3B = block_diag(W3, W3, W3, W3)
    b1t = jnp.tile(b1, 4).reshape(1, 128)
    b2t = jnp.tile(b2, 4).reshape(1, 128)

    y1p, r1p = _tc1(xp, W1lB, W1rB, b1t)
    agg1, cnt = _make_sc_segsum(True)(y1p.reshape(_NP, _D), src2d, dst2d)
    aggp1 = agg1.reshape(_NC, _PR, 128)
    cntp = cnt.reshape(_NC, _PR, 128)
    y2p, r2p = _tc2(aggp1, cntp, r1p, W2lB, W2rB, b2t)
    (agg2,) = jax.tree.leaves(
        _make_sc_segsum(False)(y2p.reshape(_NP, _D), src2d, dst2d))
    outp = _tc3(agg2.reshape(_NC, _PR, 128), cntp, r2p, W3B,
                b3.reshape(1, 1))
    return outp.reshape(_NP, 1)[:_N]
